# Initial kernel scaffold; baseline (speedup 1.0000x reference)
import jax
import jax.numpy as jnp
from jax.experimental import pallas as pl


def kernel(positions, edge_index, batch, ptr, local_or_ghost, pair_scale, e0):
    raise NotImplementedError("write your pallas kernel here")


# speedup vs baseline: 204.8328x; 204.8328x over previous
import jax
import jax.numpy as jnp
from jax import lax
from jax.experimental import pallas as pl
from jax.experimental.pallas import tpu as pltpu
from jax.experimental.pallas import tpu_sc as plsc

N_NODES = 50000
N_EDGES = 1600000
NUM_GRAPHS = 16
R2 = 25.0

NC, NS = 2, 16
NW = NC * NS
CHUNK = 128
CHUNKS_PW = 391
EPW = CHUNK * CHUNKS_PW
E_PAD = NW * EPW
DUMMY = N_NODES
POS_ROWS = 50048
PPT = POS_ROWS // NS

N_PAD = 51200
ROWS_PT = N_PAD // NS
TCB = 2048
TCG = N_PAD // TCB

NF = 13


def _sc_body(pos_hbm, src_hbm, dst_hbm, scale_hbm, z_hbm,
             acc_hbm,
             idx_s, idx_d, gbuf, vbuf, scale_v, zb,
             pos_sh, acc_sh, sem1, sem2):
    c = lax.axis_index("c")
    s = lax.axis_index("s")
    w = c * NS + s

    for f in range(4):
        pltpu.sync_copy(pos_hbm.at[pl.ds(f * POS_ROWS + s * PPT, PPT)],
                        zb.at[pl.ds(0, PPT)])
        pltpu.sync_copy(zb.at[pl.ds(0, PPT)],
                        pos_sh[f].at[pl.ds(s * PPT, PPT)])
    pltpu.sync_copy(z_hbm, zb)
    for f in range(NF):
        pltpu.sync_copy(zb, acc_sh[f].at[pl.ds(s * ROWS_PT, ROWS_PT)])
    pltpu.sync_copy(scale_hbm, scale_v)

    plsc.subcore_barrier()

    sv = scale_v[...]

    def chunk_body(i, carry):
        base = w * EPW + i * CHUNK
        pltpu.sync_copy(src_hbm.at[pl.ds(base, CHUNK)], idx_s)
        pltpu.sync_copy(dst_hbm.at[pl.ds(base, CHUNK)], idx_d)
        cps = []
        for f in range(3):
            cps.append(pltpu.async_copy(
                pos_sh[f].at[idx_s], gbuf.at[f], sem1))
        for f in range(4):
            cps.append(pltpu.async_copy(
                pos_sh[f].at[idx_d], gbuf.at[3 + f], sem1))
        for cp in cps:
            cp.wait()
        for g in range(CHUNK // 16):
            sl = pl.ds(g * 16, 16)
            xs = gbuf[0, sl]
            ys = gbuf[1, sl]
            zs = gbuf[2, sl]
            xd = gbuf[3, sl]
            yd = gbuf[4, sl]
            zd = gbuf[5, sl]
            ld = gbuf[6, sl]
            dx = xd - xs
            dy = yd - ys
            dz = zd - zs
            d2 = dx * dx + dy * dy + dz * dz
            phi = sv * jnp.exp(d2 * (-1.0 / R2))
            cphi = phi * (2.0 / R2)
            fx = cphi * dx
            fy = cphi * dy
            fz = cphi * dz
            vbuf[0, sl] = phi
            vbuf[1, sl] = fx
            vbuf[2, sl] = fy
            vbuf[3, sl] = fz
            vbuf[4, sl] = 0.5 * dx * fx
            vbuf[5, sl] = 0.5 * dx * fy
            vbuf[6, sl] = 0.5 * dx * fz
            vbuf[7, sl] = 0.5 * dy * fy
            vbuf[8, sl] = 0.5 * dy * fz
            vbuf[9, sl] = 0.5 * dz * fz
            vbuf[10, sl] = ld * fx
            vbuf[11, sl] = ld * fy
            vbuf[12, sl] = ld * fz
        cps = []
        for f in range(10):
            cps.append(pltpu.async_copy(
                vbuf.at[f], acc_sh[f].at[idx_d], sem2, add=True))
        for f in range(10, NF):
            cps.append(pltpu.async_copy(
                vbuf.at[f], acc_sh[f].at[idx_s], sem2, add=True))
        for cp in cps:
            cp.wait()
        return carry

    lax.fori_loop(0, CHUNKS_PW, chunk_body, 0)

    plsc.subcore_barrier()
    for f in range(NF):
        pltpu.sync_copy(
            acc_sh[f].at[pl.ds(s * ROWS_PT, ROWS_PT)],
            acc_hbm.at[pl.ds((c * NF + f) * N_PAD + s * ROWS_PT, ROWS_PT)])


def _tc_body(acc_ref, batch_ref, log_ref, e0_ref, nodes_ref, g_ref):
    i = pl.program_id(0)
    acc = acc_ref[0] + acc_ref[1]
    lg = log_ref[0, 0, :]
    bt = batch_ref[0, 0, :]
    e0 = e0_ref[0, 0]

    ne = acc[0, :] + e0
    fx = lg * acc[1, :] - acc[10, :]
    fy = lg * acc[2, :] - acc[11, :]
    fz = lg * acc[3, :] - acc[12, :]
    av6 = acc[4:10, :] * lg[None, :]
    nel = ne * lg

    nodes_ref[0, 0:10, :] = jnp.concatenate(
        [ne[None], fx[None], fy[None], fz[None], av6], axis=0)

    onehot = (bt[None, :] == lax.broadcasted_iota(
        jnp.int32, (NUM_GRAPHS, TCB), 0)).astype(jnp.float32)
    m8 = jnp.concatenate(
        [nel[None], av6, jnp.zeros((1, TCB), jnp.float32)], axis=0)
    pg = lax.dot_general(onehot, m8, (((1,), (1,)), ((), ())),
                         preferred_element_type=jnp.float32)

    @pl.when(i == 0)
    def _():
        g_ref[...] = jnp.zeros_like(g_ref)

    g_ref[...] += pg


def kernel(positions, edge_index, batch, ptr, local_or_ghost, pair_scale, e0):
    n = positions.shape[0]
    num_graphs = int(ptr.shape[0]) - 1

    pos4 = jnp.zeros((4, POS_ROWS), jnp.float32)
    pos4 = pos4.at[0:3, :n].set(positions.T)
    pos4 = pos4.at[3, :n].set(local_or_ghost)
    pos4 = pos4.reshape(-1)

    pad_e = E_PAD - N_EDGES
    fill = jnp.full((pad_e,), DUMMY, jnp.int32)
    srcp = jnp.concatenate([edge_index[0], fill])
    dstp = jnp.concatenate([edge_index[1], fill])

    scale16 = jnp.broadcast_to(pair_scale, (16,)).astype(jnp.float32)
    zrow = jnp.zeros((ROWS_PT,), jnp.float32)

    mesh = plsc.VectorSubcoreMesh(core_axis_name="c", subcore_axis_name="s")
    sc = pl.kernel(
        _sc_body,
        out_type=jax.ShapeDtypeStruct((NC * NF * N_PAD,), jnp.float32),
        mesh=mesh,
        scratch_types=[
            pltpu.VMEM((CHUNK,), jnp.int32),
            pltpu.VMEM((CHUNK,), jnp.int32),
            pltpu.VMEM((7, CHUNK), jnp.float32),
            pltpu.VMEM((NF, CHUNK), jnp.float32),
            pltpu.VMEM((16,), jnp.float32),
            pltpu.VMEM((ROWS_PT,), jnp.float32),
            [pltpu.VMEM_SHARED((POS_ROWS,), jnp.float32) for _ in range(4)],
            [pltpu.VMEM_SHARED((N_PAD,), jnp.float32) for _ in range(NF)],
            pltpu.SemaphoreType.DMA,
            pltpu.SemaphoreType.DMA,
        ],
    )
    acc = sc(pos4, srcp, dstp, scale16, zrow).reshape(NC, NF, N_PAD)

    batch_pad = jnp.concatenate(
        [batch, jnp.full((N_PAD - n,), NUM_GRAPHS, jnp.int32)]
    ).reshape(TCG, 1, TCB)
    log_pad = jnp.concatenate(
        [local_or_ghost, jnp.zeros((N_PAD - n,), jnp.float32)]
    ).reshape(TCG, 1, TCB)
    e0s = e0.reshape(1, 1)

    nodes, gsum = pl.pallas_call(
        _tc_body,
        grid=(TCG,),
        in_specs=[
            pl.BlockSpec((NC, NF, TCB), lambda i: (0, 0, i)),
            pl.BlockSpec((1, 1, TCB), lambda i: (i, 0, 0)),
            pl.BlockSpec((1, 1, TCB), lambda i: (i, 0, 0)),
            pl.BlockSpec(memory_space=pltpu.SMEM),
        ],
        out_specs=[
            pl.BlockSpec((1, 16, TCB), lambda i: (i, 0, 0)),
            pl.BlockSpec((NUM_GRAPHS, 8), lambda i: (0, 0)),
        ],
        out_shape=[
            jax.ShapeDtypeStruct((TCG, 16, TCB), jnp.float32),
            jax.ShapeDtypeStruct((NUM_GRAPHS, 8), jnp.float32),
        ],
    )(acc, batch_pad, log_pad, e0s)

    node_energy = nodes[:, 0, :].reshape(-1)[:n]
    forces = jnp.stack(
        [nodes[:, 1, :].reshape(-1)[:n],
         nodes[:, 2, :].reshape(-1)[:n],
         nodes[:, 3, :].reshape(-1)[:n]], axis=1)
    av = nodes[:, 4:10, :].transpose(0, 2, 1).reshape(-1, 6)[:n]

    def sym33(m6):
        return jnp.stack(
            [m6[:, 0], m6[:, 1], m6[:, 2],
             m6[:, 1], m6[:, 3], m6[:, 4],
             m6[:, 2], m6[:, 4], m6[:, 5]], axis=1).reshape(-1, 3, 3)

    atomic_virials = sym33(av)
    total_energy_local = gsum[:num_graphs, 0]
    virials = sym33(gsum[:num_graphs, 1:7])

    return (total_energy_local, node_energy, forces, virials, atomic_virials)


# speedup vs baseline: 299.5181x; 1.4623x over previous
import jax
import jax.numpy as jnp
from jax import lax
from jax.experimental import pallas as pl
from jax.experimental.pallas import tpu as pltpu
from jax.experimental.pallas import tpu_sc as plsc

N_NODES = 50000
N_EDGES = 1600000
NUM_GRAPHS = 16
R2 = 25.0

NC, NS = 2, 16
NW = NC * NS
CHUNK = 128
CHUNKS_PW = 392
EPW = CHUNK * CHUNKS_PW
E_PAD = NW * EPW
DUMMY = N_NODES
POS_ROWS = 50048
PPT = POS_ROWS // NS

N_PAD = 51200
ROWS_PT = N_PAD // NS
TCB = 2048
TCG = N_PAD // TCB

NF = 13


def _sc_body(pos_hbm, src_hbm, dst_hbm, scale_hbm, z_hbm,
             acc_hbm,
             idx_s, idx_d, gbuf, vals,
             scale_v, zb, pos_sh, acc_sh, semg, sems, semi):
    c = lax.axis_index("c")
    s = lax.axis_index("s")
    w = c * NS + s

    for f in range(4):
        pltpu.sync_copy(pos_hbm.at[pl.ds(f * POS_ROWS + s * PPT, PPT)],
                        zb.at[pl.ds(0, PPT)])
        pltpu.sync_copy(zb.at[pl.ds(0, PPT)],
                        pos_sh[f].at[pl.ds(s * PPT, PPT)])
    pltpu.sync_copy(z_hbm, zb)
    for f in range(NF):
        pltpu.sync_copy(zb, acc_sh[f].at[pl.ds(s * ROWS_PT, ROWS_PT)])
    pltpu.sync_copy(scale_hbm, scale_v)

    plsc.subcore_barrier()

    sv = scale_v[...]

    def issue_idx(i, r):
        pltpu.async_copy(src_hbm.at[pl.ds(i * CHUNK, CHUNK)],
                         idx_s[r], semi[r])
        pltpu.async_copy(dst_hbm.at[pl.ds(i * CHUNK, CHUNK)],
                         idx_d[r], semi[r])

    def wait_idx(r):
        pltpu.make_async_copy(src_hbm.at[pl.ds(0, CHUNK)],
                              idx_s[r], semi[r]).wait()
        pltpu.make_async_copy(src_hbm.at[pl.ds(0, CHUNK)],
                              idx_d[r], semi[r]).wait()

    def issue_gathers(b, r):
        for f in range(3):
            pltpu.async_copy(pos_sh[f].at[idx_s[r]], gbuf[b].at[f], semg[b])
        for f in range(4):
            pltpu.async_copy(pos_sh[f].at[idx_d[r]], gbuf[b].at[3 + f],
                             semg[b])

    def wait_gathers(b, r):
        for f in range(3):
            pltpu.make_async_copy(pos_sh[f].at[idx_s[r]], gbuf[b].at[f],
                                  semg[b]).wait()
        for f in range(4):
            pltpu.make_async_copy(pos_sh[f].at[idx_d[r]], gbuf[b].at[3 + f],
                                  semg[b]).wait()

    def issue_scatters(b, r):
        for f in range(10):
            pltpu.async_copy(vals[b].at[f], acc_sh[f].at[idx_d[r]],
                             sems[b], add=True)
        for f in range(10, NF):
            pltpu.async_copy(vals[b].at[f], acc_sh[f].at[idx_s[r]],
                             sems[b], add=True)

    def drain_scatters(b, r):
        for f in range(10):
            pltpu.make_async_copy(vals[b].at[f], acc_sh[f].at[idx_d[r]],
                                  sems[b]).wait()
        for f in range(10, NF):
            pltpu.make_async_copy(vals[b].at[f], acc_sh[f].at[idx_s[r]],
                                  sems[b]).wait()

    def compute(b):
        gb, vb = gbuf[b], vals[b]
        for g in range(CHUNK // 16):
            sl = pl.ds(g * 16, 16)
            xs = gb[0, sl]
            ys = gb[1, sl]
            zs = gb[2, sl]
            xd = gb[3, sl]
            yd = gb[4, sl]
            zd = gb[5, sl]
            ld = gb[6, sl]
            dx = xd - xs
            dy = yd - ys
            dz = zd - zs
            d2 = dx * dx + dy * dy + dz * dz
            phi = sv * jnp.exp(d2 * (-1.0 / R2))
            cphi = phi * (2.0 / R2)
            fx = cphi * dx
            fy = cphi * dy
            fz = cphi * dz
            vb[0, sl] = phi
            vb[1, sl] = fx
            vb[2, sl] = fy
            vb[3, sl] = fz
            vb[4, sl] = 0.5 * dx * fx
            vb[5, sl] = 0.5 * dx * fy
            vb[6, sl] = 0.5 * dx * fz
            vb[7, sl] = 0.5 * dy * fy
            vb[8, sl] = 0.5 * dy * fz
            vb[9, sl] = 0.5 * dz * fz
            vb[10, sl] = ld * fx
            vb[11, sl] = ld * fy
            vb[12, sl] = ld * fz

    base = w * CHUNKS_PW
    last = base + CHUNKS_PW - 1

    issue_idx(base, 0)
    issue_idx(base + 1, 1)
    wait_idx(0)
    issue_gathers(0, 0)

    def quad_body(k, carry):
        i0 = base + 4 * k
        for j in range(4):
            i = i0 + j
            ph = j % 2
            nph = (j + 1) % 2
            wait_idx((j + 1) % 4)
            issue_gathers(nph, (j + 1) % 4)
            wait_gathers(ph, j)
            if j < 2:
                @pl.when(k > 0)
                def _():
                    drain_scatters(ph, j)
            else:
                drain_scatters(ph, j)
            compute(ph)
            issue_scatters(ph, j)
            nxt = jnp.where(i + 2 <= last, i + 2, base)
            issue_idx(nxt, (j + 2) % 4)
        return carry

    lax.fori_loop(0, CHUNKS_PW // 4, quad_body, 0)

    wait_idx(1)
    wait_gathers(0, 0)
    drain_scatters(0, 2)
    drain_scatters(1, 3)

    plsc.subcore_barrier()
    for f in range(NF):
        pltpu.sync_copy(
            acc_sh[f].at[pl.ds(s * ROWS_PT, ROWS_PT)],
            acc_hbm.at[pl.ds((c * NF + f) * N_PAD + s * ROWS_PT, ROWS_PT)])


def _tc_body(acc_ref, batch_ref, log_ref, e0_ref, nodes_ref, g_ref):
    i = pl.program_id(0)
    acc = acc_ref[0] + acc_ref[1]
    lg = log_ref[0, 0, :]
    bt = batch_ref[0, 0, :]
    e0 = e0_ref[0, 0]

    ne = acc[0, :] + e0
    fx = lg * acc[1, :] - acc[10, :]
    fy = lg * acc[2, :] - acc[11, :]
    fz = lg * acc[3, :] - acc[12, :]
    av6 = acc[4:10, :] * lg[None, :]
    nel = ne * lg

    nodes_ref[0, 0:10, :] = jnp.concatenate(
        [ne[None], fx[None], fy[None], fz[None], av6], axis=0)

    onehot = (bt[None, :] == lax.broadcasted_iota(
        jnp.int32, (NUM_GRAPHS, TCB), 0)).astype(jnp.float32)
    m8 = jnp.concatenate(
        [nel[None], av6, jnp.zeros((1, TCB), jnp.float32)], axis=0)
    pg = lax.dot_general(onehot, m8, (((1,), (1,)), ((), ())),
                         preferred_element_type=jnp.float32)

    @pl.when(i == 0)
    def _():
        g_ref[...] = jnp.zeros_like(g_ref)

    g_ref[...] += pg


def kernel(positions, edge_index, batch, ptr, local_or_ghost, pair_scale, e0):
    n = positions.shape[0]
    num_graphs = int(ptr.shape[0]) - 1

    pos4 = jnp.zeros((4, POS_ROWS), jnp.float32)
    pos4 = pos4.at[0:3, :n].set(positions.T)
    pos4 = pos4.at[3, :n].set(local_or_ghost)
    pos4 = pos4.reshape(-1)

    pad_e = E_PAD - N_EDGES
    fill = jnp.full((pad_e,), DUMMY, jnp.int32)
    srcp = jnp.concatenate([edge_index[0], fill])
    dstp = jnp.concatenate([edge_index[1], fill])

    scale16 = jnp.broadcast_to(pair_scale, (16,)).astype(jnp.float32)
    zrow = jnp.zeros((ROWS_PT,), jnp.float32)

    mesh = plsc.VectorSubcoreMesh(core_axis_name="c", subcore_axis_name="s")
    sc = pl.kernel(
        _sc_body,
        out_type=jax.ShapeDtypeStruct((NC * NF * N_PAD,), jnp.float32),
        mesh=mesh,
        scratch_types=[
            [pltpu.VMEM((CHUNK,), jnp.int32) for _ in range(4)],
            [pltpu.VMEM((CHUNK,), jnp.int32) for _ in range(4)],
            [pltpu.VMEM((7, CHUNK), jnp.float32) for _ in range(2)],
            [pltpu.VMEM((NF, CHUNK), jnp.float32) for _ in range(2)],
            pltpu.VMEM((16,), jnp.float32),
            pltpu.VMEM((ROWS_PT,), jnp.float32),
            [pltpu.VMEM_SHARED((POS_ROWS,), jnp.float32) for _ in range(4)],
            [pltpu.VMEM_SHARED((N_PAD,), jnp.float32) for _ in range(NF)],
            [pltpu.SemaphoreType.DMA for _ in range(2)],
            [pltpu.SemaphoreType.DMA for _ in range(2)],
            [pltpu.SemaphoreType.DMA for _ in range(4)],
        ],
    )
    acc = sc(pos4, srcp, dstp, scale16, zrow).reshape(NC, NF, N_PAD)

    batch_pad = jnp.concatenate(
        [batch, jnp.full((N_PAD - n,), NUM_GRAPHS, jnp.int32)]
    ).reshape(TCG, 1, TCB)
    log_pad = jnp.concatenate(
        [local_or_ghost, jnp.zeros((N_PAD - n,), jnp.float32)]
    ).reshape(TCG, 1, TCB)
    e0s = e0.reshape(1, 1)

    nodes, gsum = pl.pallas_call(
        _tc_body,
        grid=(TCG,),
        in_specs=[
            pl.BlockSpec((NC, NF, TCB), lambda i: (0, 0, i)),
            pl.BlockSpec((1, 1, TCB), lambda i: (i, 0, 0)),
            pl.BlockSpec((1, 1, TCB), lambda i: (i, 0, 0)),
            pl.BlockSpec(memory_space=pltpu.SMEM),
        ],
        out_specs=[
            pl.BlockSpec((1, 16, TCB), lambda i: (i, 0, 0)),
            pl.BlockSpec((NUM_GRAPHS, 8), lambda i: (0, 0)),
        ],
        out_shape=[
            jax.ShapeDtypeStruct((TCG, 16, TCB), jnp.float32),
            jax.ShapeDtypeStruct((NUM_GRAPHS, 8), jnp.float32),
        ],
    )(acc, batch_pad, log_pad, e0s)

    node_energy = nodes[:, 0, :].reshape(-1)[:n]
    forces = jnp.stack(
        [nodes[:, 1, :].reshape(-1)[:n],
         nodes[:, 2, :].reshape(-1)[:n],
         nodes[:, 3, :].reshape(-1)[:n]], axis=1)
    av = nodes[:, 4:10, :].transpose(0, 2, 1).reshape(-1, 6)[:n]

    def sym33(m6):
        return jnp.stack(
            [m6[:, 0], m6[:, 1], m6[:, 2],
             m6[:, 1], m6[:, 3], m6[:, 4],
             m6[:, 2], m6[:, 4], m6[:, 5]], axis=1).reshape(-1, 3, 3)

    atomic_virials = sym33(av)
    total_energy_local = gsum[:num_graphs, 0]
    virials = sym33(gsum[:num_graphs, 1:7])

    return (total_energy_local, node_energy, forces, virials, atomic_virials)
